# Initial kernel scaffold; baseline (speedup 1.0000x reference)
import jax
import jax.numpy as jnp
from jax.experimental import pallas as pl


def kernel(feat):
    raise NotImplementedError("write your pallas kernel here")


# speedup vs baseline: 20.3276x; 20.3276x over previous
import numpy as np
import jax
import jax.numpy as jnp
from jax.experimental import pallas as pl
from jax.experimental.pallas import tpu as pltpu

_NUM_ANGLE = 100
_NUM_RHO = 100
_H = 100
_W = 100
_P = _H * _W
_P_PAD = 10240
_A_PAD = 104
_A_BLK = 8
_P_BLK = 2048


def _rk_table() -> np.ndarray:
    irho = float(int(np.sqrt(_H * _H + _W * _W) + 1)) / float(_NUM_RHO - 1)
    itheta = np.pi / _NUM_ANGLE
    angles = np.arange(_NUM_ANGLE, dtype=np.float64) * itheta
    tab_cos = (np.cos(angles) / irho).astype(np.float32)
    tab_sin = (np.sin(angles) / irho).astype(np.float32)
    ys, xs = np.meshgrid(np.arange(_H), np.arange(_W), indexing="ij")
    xx = (xs - (_W // 2)).reshape(-1).astype(np.float32)
    yy = (ys - (_H // 2)).reshape(-1).astype(np.float32)
    proj = xx[None, :] * tab_cos[:, None] + yy[None, :] * tab_sin[:, None]
    r = np.where(proj >= 0,
                 np.floor(proj + np.float32(0.5)),
                 np.ceil(proj - np.float32(0.5))).astype(np.int32) + _NUM_RHO // 2
    r = np.clip(r, 0, _NUM_RHO - 1)
    rk = r + (np.arange(_NUM_ANGLE, dtype=np.int32) * _NUM_RHO)[:, None]
    out = np.full((_A_PAD, _P_PAD), -1, dtype=np.int32)
    out[:_NUM_ANGLE, :_P] = rk
    return out


_RK = _rk_table()


def _hough_body(rk_ref, f_ref, o_ref):
    i = pl.program_id(0)
    j = pl.program_id(1)

    @pl.when(j == 0)
    def _init():
        o_ref[...] = jnp.zeros_like(o_ref)

    rk = rk_ref[...]
    rk_e = jnp.broadcast_to(
        rk[:, None, :], (_A_BLK, _NUM_RHO, _P_BLK)
    ).reshape(_A_BLK * _NUM_RHO, _P_BLK)
    kcol = i * (_A_BLK * _NUM_RHO) + jax.lax.broadcasted_iota(
        jnp.int32, (_A_BLK * _NUM_RHO, _P_BLK), 0)
    oh = (rk_e == kcol).astype(jnp.bfloat16)
    f = f_ref[...]
    acc = jax.lax.dot_general(
        f, oh, (((1,), (1,)), ((), ())),
        preferred_element_type=jnp.float32)
    o_ref[...] += acc.reshape(o_ref.shape)


def kernel(feat):
    n, c, h, w = feat.shape
    nc = n * c
    feat2d = feat.reshape(nc, _P).astype(jnp.bfloat16)
    feat2d = jnp.pad(feat2d, ((0, 0), (0, _P_PAD - _P)))
    rk = jnp.asarray(_RK)

    out = pl.pallas_call(
        _hough_body,
        grid=(_A_PAD // _A_BLK, _P_PAD // _P_BLK),
        in_specs=[
            pl.BlockSpec((_A_BLK, _P_BLK), lambda i, j: (i, j)),
            pl.BlockSpec((nc, _P_BLK), lambda i, j: (0, j)),
        ],
        out_specs=pl.BlockSpec((nc, _A_BLK, _NUM_RHO), lambda i, j: (0, i, 0)),
        out_shape=jax.ShapeDtypeStruct((nc, _A_PAD, _NUM_RHO), jnp.float32),
        compiler_params=pltpu.CompilerParams(
            dimension_semantics=("arbitrary", "arbitrary"),
        ),
    )(rk, feat2d)

    out = out[:, : _NUM_ANGLE, :]
    return out.reshape(n, c, _NUM_ANGLE, _NUM_RHO)


# speedup vs baseline: 23.4433x; 1.1533x over previous
import numpy as np
import jax
import jax.numpy as jnp
from jax.experimental import pallas as pl
from jax.experimental.pallas import tpu as pltpu

_NUM_ANGLE = 100
_NUM_RHO = 100
_H = 100
_W = 100
_P = _H * _W
_A_BLK = 8
_A_STEPS = 13


def _rk_table() -> np.ndarray:
    irho = float(int(np.sqrt(_H * _H + _W * _W) + 1)) / float(_NUM_RHO - 1)
    itheta = np.pi / _NUM_ANGLE
    angles = np.arange(_NUM_ANGLE, dtype=np.float64) * itheta
    tab_cos = (np.cos(angles) / irho).astype(np.float32)
    tab_sin = (np.sin(angles) / irho).astype(np.float32)
    ys, xs = np.meshgrid(np.arange(_H), np.arange(_W), indexing="ij")
    xx = (xs - (_W // 2)).reshape(-1).astype(np.float32)
    yy = (ys - (_H // 2)).reshape(-1).astype(np.float32)
    proj = xx[None, :] * tab_cos[:, None] + yy[None, :] * tab_sin[:, None]
    r = np.where(proj >= 0,
                 np.floor(proj + np.float32(0.5)),
                 np.ceil(proj - np.float32(0.5))).astype(np.int32) + _NUM_RHO // 2
    r = np.clip(r, 0, _NUM_RHO - 1)
    rk = r + (np.arange(_NUM_ANGLE, dtype=np.int32) * _NUM_RHO)[:, None]
    out = np.full((_A_STEPS * _A_BLK, _P), -1, dtype=np.int32)
    out[:_NUM_ANGLE] = rk
    return out


_RK = _rk_table()


def _hough_body(rk_ref, f_ref, o_ref):
    i = pl.program_id(0)
    rk = rk_ref[...]
    rk_e = jnp.broadcast_to(
        rk[:, None, :], (_A_BLK, _NUM_RHO, _P)
    ).reshape(_A_BLK * _NUM_RHO, _P)
    kcol = i * (_A_BLK * _NUM_RHO) + jax.lax.broadcasted_iota(
        jnp.int32, (_A_BLK * _NUM_RHO, _P), 0)
    oh = (rk_e == kcol).astype(jnp.bfloat16)
    f = f_ref[...]
    acc = jax.lax.dot_general(
        f, oh, (((1,), (1,)), ((), ())),
        preferred_element_type=jnp.float32)
    o_ref[...] = acc.reshape(o_ref.shape)


def kernel(feat):
    n, c, h, w = feat.shape
    nc = n * c
    feat2d = feat.reshape(nc, _P).astype(jnp.bfloat16)
    rk = jnp.asarray(_RK)

    out = pl.pallas_call(
        _hough_body,
        grid=(_A_STEPS,),
        in_specs=[
            pl.BlockSpec((_A_BLK, _P), lambda i: (i, 0)),
            pl.BlockSpec((nc, _P), lambda i: (0, 0)),
        ],
        out_specs=pl.BlockSpec((nc, _A_BLK, _NUM_RHO), lambda i: (0, i, 0)),
        out_shape=jax.ShapeDtypeStruct((nc, _NUM_ANGLE, _NUM_RHO), jnp.float32),
        compiler_params=pltpu.CompilerParams(
            dimension_semantics=("arbitrary",),
        ),
    )(rk, feat2d)

    return out.reshape(n, c, _NUM_ANGLE, _NUM_RHO)
